# baseline (speedup 1.0000x reference)
import functools

import jax
import jax.numpy as jnp
from jax import lax
from jax.experimental import pallas as pl
from jax.experimental.pallas import tpu as pltpu
from jax.experimental.pallas import tpu_sc as plsc

EMBED = 32
LANES = 16
CHUNK = 128


@functools.cache
def _build_sc(batch):
    info = plsc.get_sparse_core_info()
    nw = info.num_cores * info.num_subcores
    bpw = batch // nw
    nchunk = bpw // CHUNK
    mesh = plsc.VectorSubcoreMesh(core_axis_name="c", subcore_axis_name="s")

    @functools.partial(
        pl.kernel,
        mesh=mesh,
        compiler_params=pltpu.CompilerParams(use_tc_tiling_on_sc=False),
        out_type=jax.ShapeDtypeStruct((nw, bpw, EMBED), jnp.float32),
        scratch_types=[
            pltpu.VMEM((nchunk, CHUNK), jnp.int32),
            pltpu.VMEM((nchunk, CHUNK), jnp.int32),
            pltpu.VMEM((bpw, EMBED), jnp.float32),
            pltpu.VMEM((bpw, EMBED), jnp.float32),
            pltpu.SemaphoreType.DMA,
        ],
    )
    def gather_mul(prod_hbm, user_hbm, ptab_hbm, utab_hbm, p_hbm,
                   pidx_v, uidx_v, arows_v, brows_v, sem):
        wid = lax.axis_index("s") * info.num_cores + lax.axis_index("c")
        pltpu.sync_copy(prod_hbm.at[wid], pidx_v)
        pltpu.sync_copy(user_hbm.at[wid], uidx_v)
        handles = []
        for j in range(nchunk):
            handles.append(pltpu.async_copy(
                ptab_hbm.at[pidx_v.at[j]], arows_v.at[pl.ds(j * CHUNK, CHUNK)], sem))
            handles.append(pltpu.async_copy(
                utab_hbm.at[uidx_v.at[j]], brows_v.at[pl.ds(j * CHUNK, CHUNK)], sem))
        for h in handles:
            h.wait()

        def mul_row(r, carry):
            for h in range(EMBED // LANES):
                sl = pl.ds(h * LANES, LANES)
                arows_v[r, sl] = arows_v[r, sl] * brows_v[r, sl]
            return carry

        lax.fori_loop(0, bpw, mul_row, 0)
        pltpu.sync_copy(arows_v, p_hbm.at[wid])

    return gather_mul


def _tc_reduce_body(p_ref, o_ref):
    o_ref[...] = jnp.sum(p_ref[...], axis=1)


@functools.cache
def _build_tc(batch):
    rows_per_block = 2048
    grid = batch // rows_per_block
    return pl.pallas_call(
        _tc_reduce_body,
        grid=(grid,),
        in_specs=[pl.BlockSpec((rows_per_block, EMBED), lambda i: (i, 0))],
        out_specs=pl.BlockSpec((rows_per_block,), lambda i: (i,)),
        out_shape=jax.ShapeDtypeStruct((batch,), jnp.float32),
    )


def kernel(products, users, product_embedding, user_embedding):
    batch = products.shape[0]
    info = plsc.get_sparse_core_info()
    nw = info.num_cores * info.num_subcores
    nchunk = batch // nw // CHUNK
    prod2 = products.astype(jnp.int32).reshape(nw, nchunk, CHUNK)
    user2 = users.astype(jnp.int32).reshape(nw, nchunk, CHUNK)
    p = _build_sc(batch)(prod2, user2, product_embedding, user_embedding)
    return _build_tc(batch)(p.reshape(batch, EMBED))
